# baseline (speedup 1.0000x reference)
import functools

import jax
import jax.numpy as jnp
from jax import lax
from jax.experimental import pallas as pl
from jax.experimental.pallas import tpu as pltpu
from jax.experimental.pallas import tpu_sc as plsc

B = 16384
NUM_ASPECTS = 5
H1 = 128
D = NUM_ASPECTS * H1

_NC = 2
_NS = 16
_NW = _NC * _NS
_ROWS = 128
_COLS = 128
_RPW = _ROWS // _NW


def _sc_offsets_body(uid_hbm, iid_hbm, utab_hbm, itab_hbm, out_hbm,
                     uidx, iidx, urow, irow, comb, sem):
    wid = lax.axis_index("s") * _NC + lax.axis_index("c")
    base = wid * _RPW
    pltpu.sync_copy(uid_hbm.at[pl.ds(base, _RPW)], uidx)
    pltpu.sync_copy(iid_hbm.at[pl.ds(base, _RPW)], iidx)
    handles = []
    for j in range(_RPW):
        handles.append(pltpu.async_copy(utab_hbm.at[uidx.at[j]], urow.at[j], sem))
        handles.append(pltpu.async_copy(itab_hbm.at[iidx.at[j]], irow.at[j], sem))
    for h in handles:
        h.wait()
    for j in range(_RPW):
        for k in range(_COLS // 16):
            sl = pl.ds(k * 16, 16)
            comb[j, sl] = urow[j, sl] + irow[j, sl]
    pltpu.sync_copy(comb, out_hbm.at[pl.ds(base, _RPW)])


def _sc_offsets(uid2, iid2, utab, itab):
    mesh = plsc.VectorSubcoreMesh(core_axis_name="c", subcore_axis_name="s")
    kern = functools.partial(
        pl.kernel,
        mesh=mesh,
        out_type=jax.ShapeDtypeStruct((_ROWS, _COLS), jnp.float32),
        scratch_types=[
            pltpu.VMEM((_RPW, _COLS), jnp.int32),
            pltpu.VMEM((_RPW, _COLS), jnp.int32),
            pltpu.VMEM((_RPW, _COLS), jnp.float32),
            pltpu.VMEM((_RPW, _COLS), jnp.float32),
            pltpu.VMEM((_RPW, _COLS), jnp.float32),
            pltpu.SemaphoreType.DMA,
        ],
    )(_sc_offsets_body)
    return kern(uid2, iid2, utab, itab)


_RB = 1024
_NB = B // _RB


def _tc_body(g_ref, u_ref, v_ref, c_ref, o_ref):
    s = jnp.sum(u_ref[...] * v_ref[...], axis=1)
    o_ref[...] = (s + c_ref[0, 0, :] + g_ref[0, 0]).reshape(1, 1, _RB)


def _tc_rating(u2, v2, comb3, g2):
    return pl.pallas_call(
        _tc_body,
        grid=(_NB,),
        in_specs=[
            pl.BlockSpec(memory_space=pltpu.SMEM),
            pl.BlockSpec((_RB, D), lambda i: (i, 0)),
            pl.BlockSpec((_RB, D), lambda i: (i, 0)),
            pl.BlockSpec((1, 1, _RB), lambda i: (i, 0, 0)),
        ],
        out_specs=pl.BlockSpec((1, 1, _RB), lambda i: (i, 0, 0)),
        out_shape=jax.ShapeDtypeStruct((_NB, 1, _RB), jnp.float32),
    )(g2, u2, v2, comb3)


def kernel(userAspRep, itemAspRep, batch_uid, batch_iid, user_offset,
           item_offset, global_offset):
    uid2 = batch_uid.reshape(_ROWS, _COLS)
    iid2 = batch_iid.reshape(_ROWS, _COLS)
    utab = user_offset.reshape(-1)
    itab = item_offset.reshape(-1)
    comb = _sc_offsets(uid2, iid2, utab, itab)

    u2 = userAspRep.reshape(B, D)
    v2 = itemAspRep.reshape(B, D)
    comb3 = comb.reshape(_NB, 1, _RB)
    g2 = global_offset.reshape(1, 1)
    out = _tc_rating(u2, v2, comb3, g2)
    return out.reshape(B, 1)
